# baseline (speedup 1.0000x reference)
import functools

import jax
import jax.numpy as jnp
from jax import lax
from jax.experimental import pallas as pl
from jax.experimental.pallas import tpu as pltpu
from jax.experimental.pallas import tpu_sc as plsc

_LANES = 128


def _make_gather(num_rows: int, num_ids_rows: int, num_ids_cols: int, dim: int):
    info = plsc.get_sparse_core_info()
    nc, ns = info.num_cores, info.num_subcores
    nl = info.num_lanes
    nw = nc * ns
    batch = num_ids_rows * num_ids_cols
    b_per_w = batch // nw
    per_row = num_ids_cols // b_per_w
    mesh = plsc.VectorSubcoreMesh(core_axis_name="c", subcore_axis_name="s")

    @functools.partial(
        pl.kernel,
        mesh=mesh,
        compiler_params=pltpu.CompilerParams(needs_layout_passes=False),
        out_type=jax.ShapeDtypeStruct((num_ids_rows, dim, num_ids_cols), jnp.float32),
        scratch_types=[
            pltpu.VMEM((b_per_w,), jnp.int32),
            pltpu.VMEM((b_per_w, _LANES), jnp.float32),
            pltpu.VMEM((dim, b_per_w), jnp.float32),
            pltpu.SemaphoreType.DMA,
        ],
    )
    def gather_kernel(table_hbm, idx_hbm, out_hbm, idx_v, rows_w, rows_t, sem):
        wid = lax.axis_index("s") * nc + lax.axis_index("c")
        r = wid // per_row
        col = (wid % per_row) * b_per_w
        pltpu.sync_copy(idx_hbm.at[r, pl.ds(col, b_per_w)], idx_v)
        pltpu.async_copy(table_hbm.at[idx_v], rows_w, sem).wait()

        lane = lax.iota(jnp.int32, nl)

        def transpose_d(d, carry):
            d_vec = jnp.full((nl,), 0, jnp.int32) + d
            for c16 in range(b_per_w // nl):
                c_vec = c16 * nl + lane
                rows_t[d, pl.ds(c16 * nl, nl)] = plsc.load_gather(
                    rows_w, [c_vec, d_vec]
                )
            return carry

        lax.fori_loop(0, dim, transpose_d, 0)
        pltpu.sync_copy(rows_t, out_hbm.at[r, :, pl.ds(col, b_per_w)])

    return gather_kernel


def kernel(embed_id, weight):
    num_rows, dim = weight.shape
    ir, ic = embed_id.shape
    wpad = jnp.pad(weight, ((0, 0), (0, _LANES - dim)))
    out_t = _make_gather(num_rows, ir, ic, dim)(wpad, embed_id.astype(jnp.int32))
    return out_t.transpose(0, 2, 1)


# speedup vs baseline: 1.3061x; 1.3061x over previous
import functools

import jax
import jax.numpy as jnp
from jax import lax
from jax.experimental import pallas as pl
from jax.experimental.pallas import tpu as pltpu
from jax.experimental.pallas import tpu_sc as plsc

_LANES = 128


def _make_gather(num_rows: int, num_ids_rows: int, num_ids_cols: int, dim: int):
    info = plsc.get_sparse_core_info()
    nc, ns = info.num_cores, info.num_subcores
    nl = info.num_lanes
    nw = nc * ns
    batch = num_ids_rows * num_ids_cols
    b_per_w = batch // nw
    per_row = num_ids_cols // b_per_w
    mesh = plsc.VectorSubcoreMesh(core_axis_name="c", subcore_axis_name="s")

    @functools.partial(
        pl.kernel,
        mesh=mesh,
        compiler_params=pltpu.CompilerParams(needs_layout_passes=False),
        out_type=jax.ShapeDtypeStruct((num_ids_rows, dim, num_ids_cols), jnp.float32),
        scratch_types=[
            pltpu.VMEM((b_per_w,), jnp.int32),
            pltpu.VMEM((b_per_w, _LANES), jnp.float32),
            pltpu.VMEM((dim, b_per_w), jnp.float32),
            pltpu.SemaphoreType.DMA,
        ],
    )
    def gather_kernel(table_hbm, idx_hbm, out_hbm, idx_v, rows_w, rows_t, sem):
        wid = lax.axis_index("s") * nc + lax.axis_index("c")
        r = wid // per_row
        col = (wid % per_row) * b_per_w
        pltpu.sync_copy(idx_hbm.at[r, pl.ds(col, b_per_w)], idx_v)
        pltpu.async_copy(table_hbm.at[idx_v], rows_w, sem).wait()

        lane = lax.iota(jnp.int32, nl)
        diag = [(lane + k) & (nl - 1) for k in range(nl)]

        def transpose_cblk(cb, carry):
            c_vec = cb * nl + lane
            for db in range(dim // nl):
                for k in range(nl):
                    d_vec = db * nl + diag[k]
                    v = plsc.load_gather(rows_w, [c_vec, d_vec])
                    plsc.store_scatter(rows_t, [d_vec, c_vec], v)
            return carry

        lax.fori_loop(0, b_per_w // nl, transpose_cblk, 0)
        pltpu.sync_copy(rows_t, out_hbm.at[r, :, pl.ds(col, b_per_w)])

    return gather_kernel


def kernel(embed_id, weight):
    num_rows, dim = weight.shape
    ir, ic = embed_id.shape
    wpad = jnp.pad(weight, ((0, 0), (0, _LANES - dim)))
    out_t = _make_gather(num_rows, ir, ic, dim)(wpad, embed_id.astype(jnp.int32))
    return out_t.transpose(0, 2, 1)
